# Initial kernel scaffold; baseline (speedup 1.0000x reference)
import jax
import jax.numpy as jnp
from jax.experimental import pallas as pl


def kernel(hidden_states, Wg_sh, Wu_sh, Wd_sh, W_router, e_bias, We_gate, We_up, We_down):
    raise NotImplementedError("write your pallas kernel here")


# speedup vs baseline: 1.5110x; 1.5110x over previous
import functools

import jax
import jax.numpy as jnp
from jax.experimental import pallas as pl
from jax.experimental.pallas import tpu as pltpu

T = 4096
D = 1024
I = 512
E = 16
TOP_K = 2
N_GROUP = 4
TOPK_GROUP = 2
GROUP_SIZE = E // N_GROUP
ROUTED_SCALING_FACTOR = 2.5

_BT_ROUTE = 512
_BT_MOE = 2048


def _lane_roll(x, s):
    return jnp.concatenate([x[:, s:], x[:, :s]], axis=1)


def _routing_body(x_ref, wr_ref, bias_ref, comb_ref):
    x = x_ref[...]
    logits = jnp.dot(x, wr_ref[...], preferred_element_type=jnp.float32)
    scores = jax.nn.sigmoid(logits)
    sfc = scores + bias_ref[...]

    gs_cols = []
    for g in range(N_GROUP):
        a = sfc[:, 4 * g + 0 : 4 * g + 1]
        b = sfc[:, 4 * g + 1 : 4 * g + 2]
        c = sfc[:, 4 * g + 2 : 4 * g + 3]
        d = sfc[:, 4 * g + 3 : 4 * g + 4]
        hi1 = jnp.maximum(a, b)
        lo1 = jnp.minimum(a, b)
        hi2 = jnp.maximum(c, d)
        lo2 = jnp.minimum(c, d)
        top1 = jnp.maximum(hi1, hi2)
        top2 = jnp.maximum(jnp.minimum(hi1, hi2), jnp.maximum(lo1, lo2))
        gs_cols.append(top1 + top2)

    grank = [jnp.zeros_like(gs_cols[0]) for _ in range(N_GROUP)]
    for g in range(N_GROUP):
        for g2 in range(N_GROUP):
            if g2 == g:
                continue
            beats = (gs_cols[g2] >= gs_cols[g]) if g2 < g else (gs_cols[g2] > gs_cols[g])
            grank[g] = grank[g] + beats.astype(jnp.float32)
    gmask = [grank[g] < TOPK_GROUP for g in range(N_GROUP)]
    ms = jnp.concatenate(
        [jnp.where(gmask[g], sfc[:, 4 * g : 4 * g + 4], 0.0) for g in range(N_GROUP)],
        axis=1,
    )

    lane = jax.lax.broadcasted_iota(jnp.int32, ms.shape, 1)
    rank = jnp.zeros_like(ms)
    for s in range(1, E):
        rolled = _lane_roll(ms, s)
        wraps = lane >= (E - s)
        beats = (rolled > ms) | ((rolled == ms) & wraps)
        rank = rank + beats.astype(jnp.float32)
    chosen = rank < TOP_K

    wsum = jnp.sum(jnp.where(chosen, scores, 0.0), axis=1, keepdims=True) + 1e-20
    cw = jnp.where(chosen, scores / wsum, 0.0) * ROUTED_SCALING_FACTOR
    ones = jnp.ones_like(cw[:, :1])
    comb_ref[...] = jnp.concatenate([cw, ones], axis=1)


def _routing(x, w_router, e_bias, interpret=False):
    return pl.pallas_call(
        _routing_body,
        grid=(T // _BT_ROUTE,),
        in_specs=[
            pl.BlockSpec((_BT_ROUTE, D), lambda i: (i, 0)),
            pl.BlockSpec((D, E), lambda i: (0, 0)),
            pl.BlockSpec((1, E), lambda i: (0, 0)),
        ],
        out_specs=pl.BlockSpec((_BT_ROUTE, E + 1), lambda i: (i, 0)),
        out_shape=jax.ShapeDtypeStruct((T, E + 1), jnp.float32),
        interpret=interpret,
    )(x, w_router, e_bias.reshape(1, E))


def _moe_body(x_ref, wg_ref, wu_ref, wd_ref, comb_ref, out_ref, acc_ref):
    j = pl.program_id(1)
    x = x_ref[...]
    g = jnp.dot(x, wg_ref[0], preferred_element_type=jnp.float32)
    u = jnp.dot(x, wu_ref[0], preferred_element_type=jnp.float32)
    h = (g * jax.nn.sigmoid(g) * u).astype(jnp.bfloat16)
    y = jnp.dot(h, wd_ref[0], preferred_element_type=jnp.float32)
    contrib = y * comb_ref[0]

    @pl.when(j == 0)
    def _():
        acc_ref[...] = contrib

    @pl.when(j > 0)
    def _():
        acc_ref[...] = acc_ref[...] + contrib

    @pl.when(j == E)
    def _():
        out_ref[...] = acc_ref[...]


def _moe(xb, wg_all, wu_all, wd_all, comb3, interpret=False):
    nt = T // _BT_MOE
    return pl.pallas_call(
        _moe_body,
        grid=(nt, E + 1),
        in_specs=[
            pl.BlockSpec((_BT_MOE, D), lambda i, j: (i, 0)),
            pl.BlockSpec((1, D, I), lambda i, j: (j, 0, 0)),
            pl.BlockSpec((1, D, I), lambda i, j: (j, 0, 0)),
            pl.BlockSpec((1, I, D), lambda i, j: (j, 0, 0)),
            pl.BlockSpec((1, _BT_MOE, 1), lambda i, j: (j, i, 0)),
        ],
        out_specs=pl.BlockSpec((_BT_MOE, D), lambda i, j: (i, 0)),
        out_shape=jax.ShapeDtypeStruct((T, D), jnp.float32),
        scratch_shapes=[pltpu.VMEM((_BT_MOE, D), jnp.float32)],
        interpret=interpret,
    )(xb, wg_all, wu_all, wd_all, comb3)


def _kernel_impl(hidden_states, Wg_sh, Wu_sh, Wd_sh, W_router, e_bias,
                 We_gate, We_up, We_down, interpret=False):
    comb = _routing(hidden_states, W_router, e_bias, interpret=interpret)
    comb3 = comb.T.reshape(E + 1, T, 1)
    xb = hidden_states.astype(jnp.bfloat16)
    wg_all = jnp.concatenate([We_gate, Wg_sh[None]], axis=0).astype(jnp.bfloat16)
    wu_all = jnp.concatenate([We_up, Wu_sh[None]], axis=0).astype(jnp.bfloat16)
    wd_all = jnp.concatenate([We_down, Wd_sh[None]], axis=0).astype(jnp.bfloat16)
    return _moe(xb, wg_all, wu_all, wd_all, comb3, interpret=interpret)


def kernel(hidden_states, Wg_sh, Wu_sh, Wd_sh, W_router, e_bias, We_gate, We_up, We_down):
    return _kernel_impl(hidden_states, Wg_sh, Wu_sh, Wd_sh, W_router, e_bias,
                        We_gate, We_up, We_down)


# speedup vs baseline: 2.2871x; 1.5136x over previous
import functools

import jax
import jax.numpy as jnp
from jax import lax
from jax.experimental import pallas as pl
from jax.experimental.pallas import tpu as pltpu
from jax.experimental.pallas import tpu_sc as plsc

T = 4096
D = 1024
I = 512
E = 16
TOP_K = 2
N_GROUP = 4
TOPK_GROUP = 2
ROUTED_SCALING_FACTOR = 2.5

S = T * TOP_K
_BT_ROUTE = 512
_NB = T // _BT_ROUTE
_TM = 256
_NT = S // _TM
_G = _NT + E - 1
_BT_FIN = 1024

_NC = 2
_NS = 16
_NW = _NC * _NS
_CHT = T // _NW


def _lane_roll(x, s):
    return jnp.concatenate([x[:, s:], x[:, :s]], axis=1)


def _cumsum_rows_excl(x):
    inc = x
    s = 1
    while s < x.shape[0]:
        inc = inc + jnp.concatenate(
            [jnp.zeros((s, inc.shape[1]), inc.dtype), inc[:-s]], axis=0
        )
        s *= 2
    return inc - x


def _routing_body(x_ref, wr_ref, bias_ref, ids_ref, w_ref, cnt_ref, rk_ref):
    x = x_ref[...]
    logits = jnp.dot(x, wr_ref[...], preferred_element_type=jnp.float32)
    scores = jax.nn.sigmoid(logits)
    sfc = scores + bias_ref[...]

    gs_cols = []
    for g in range(N_GROUP):
        a = sfc[:, 4 * g + 0 : 4 * g + 1]
        b = sfc[:, 4 * g + 1 : 4 * g + 2]
        c = sfc[:, 4 * g + 2 : 4 * g + 3]
        d = sfc[:, 4 * g + 3 : 4 * g + 4]
        hi1, lo1 = jnp.maximum(a, b), jnp.minimum(a, b)
        hi2, lo2 = jnp.maximum(c, d), jnp.minimum(c, d)
        top1 = jnp.maximum(hi1, hi2)
        top2 = jnp.maximum(jnp.minimum(hi1, hi2), jnp.maximum(lo1, lo2))
        gs_cols.append(top1 + top2)

    grank = [jnp.zeros_like(gs_cols[0]) for _ in range(N_GROUP)]
    for g in range(N_GROUP):
        for g2 in range(N_GROUP):
            if g2 == g:
                continue
            beats = (gs_cols[g2] >= gs_cols[g]) if g2 < g else (gs_cols[g2] > gs_cols[g])
            grank[g] = grank[g] + beats.astype(jnp.float32)
    gmask = [grank[g] < TOPK_GROUP for g in range(N_GROUP)]
    ms = jnp.concatenate(
        [jnp.where(gmask[g], sfc[:, 4 * g : 4 * g + 4], 0.0) for g in range(N_GROUP)],
        axis=1,
    )

    lane = jax.lax.broadcasted_iota(jnp.int32, ms.shape, 1)
    rank = jnp.zeros_like(ms)
    for s in range(1, E):
        rolled = _lane_roll(ms, s)
        wraps = lane >= (E - s)
        beats = (rolled > ms) | ((rolled == ms) & wraps)
        rank = rank + beats.astype(jnp.float32)
    chosen = rank < TOP_K
    chf = chosen.astype(jnp.float32)

    wsum = jnp.sum(jnp.where(chosen, scores, 0.0), axis=1, keepdims=True) + 1e-20
    cw = (scores / wsum) * ROUTED_SCALING_FACTOR

    lanef = lane.astype(jnp.float32)
    id0 = jnp.min(jnp.where(chosen, lanef, 1e9), axis=1, keepdims=True)
    id1 = jnp.max(jnp.where(chosen, lanef, -1.0), axis=1, keepdims=True)
    is0 = lanef == id0
    is1 = lanef == id1
    w0 = jnp.sum(jnp.where(is0, cw, 0.0), axis=1, keepdims=True)
    w1 = jnp.sum(jnp.where(is1, cw, 0.0), axis=1, keepdims=True)

    cum = _cumsum_rows_excl(chf)
    r0 = jnp.sum(jnp.where(is0, cum, 0.0), axis=1, keepdims=True)
    r1 = jnp.sum(jnp.where(is1, cum, 0.0), axis=1, keepdims=True)

    ids_ref[...] = jnp.concatenate([id0, id1], axis=1).astype(jnp.int32)
    w_ref[...] = jnp.concatenate([w0, w1], axis=1)
    cnt_ref[...] = jnp.sum(chf, axis=0, keepdims=True).astype(jnp.int32)[None]
    rk_ref[...] = jnp.concatenate([r0, r1], axis=1).astype(jnp.int32)


def _routing(x, w_router, e_bias, interpret=False):
    return pl.pallas_call(
        _routing_body,
        grid=(_NB,),
        in_specs=[
            pl.BlockSpec((_BT_ROUTE, D), lambda i: (i, 0)),
            pl.BlockSpec((D, E), lambda i: (0, 0)),
            pl.BlockSpec((1, E), lambda i: (0, 0)),
        ],
        out_specs=[
            pl.BlockSpec((_BT_ROUTE, 2), lambda i: (i, 0)),
            pl.BlockSpec((_BT_ROUTE, 2), lambda i: (i, 0)),
            pl.BlockSpec((1, 1, E), lambda i: (i, 0, 0)),
            pl.BlockSpec((_BT_ROUTE, 2), lambda i: (i, 0)),
        ],
        out_shape=[
            jax.ShapeDtypeStruct((T, 2), jnp.int32),
            jax.ShapeDtypeStruct((T, 2), jnp.float32),
            jax.ShapeDtypeStruct((_NB, 1, E), jnp.int32),
            jax.ShapeDtypeStruct((T, 2), jnp.int32),
        ],
        interpret=interpret,
    )(x, w_router, e_bias.reshape(1, E))


def _dispatch_meta(cnt_blk):
    tot = jnp.sum(cnt_blk, axis=0)
    offs = jnp.concatenate([jnp.zeros((1,), jnp.int32), jnp.cumsum(tot)]).astype(jnp.int32)
    blk_excl = jnp.cumsum(cnt_blk, axis=0) - cnt_blk
    base_flat = (offs[:E][None, :] + blk_excl).reshape(_NB * E).astype(jnp.int32)

    first_tile = offs[:E] // _TM
    ntiles = jnp.where(tot > 0, -(-offs[1:] // _TM) - first_tile, 0).astype(jnp.int32)
    start = (jnp.cumsum(ntiles) - ntiles).astype(jnp.int32)
    total = jnp.sum(ntiles)
    i = jnp.arange(_G, dtype=jnp.int32)
    se = jnp.sum((start[None, :] + ntiles[None, :]) <= i[:, None], axis=1).astype(jnp.int32)
    se = jnp.clip(se, 0, E - 1)
    st = jnp.take(first_tile, se) + (i - jnp.take(start, se))
    valid = i < total
    st = jnp.where(valid, st, _NT - 1).astype(jnp.int32)
    lo = jnp.where(valid, jnp.maximum(jnp.take(offs, se), st * _TM), 0).astype(jnp.int32)
    hi = jnp.where(valid, jnp.minimum(jnp.take(offs, se + 1), (st + 1) * _TM), 0).astype(jnp.int32)
    fi = jnp.concatenate(
        [jnp.ones((1,), jnp.int32), (st[1:] != st[:-1]).astype(jnp.int32)]
    )
    return base_flat, se, st, fi, lo, hi


def _pos_body(ids_ref, rk_ref, base_ref, pos0_ref, pos1_ref):
    ids = ids_ref[...]
    rk = rk_ref[...].astype(jnp.float32)
    base_row = base_ref[0].astype(jnp.float32)
    iota = jax.lax.broadcasted_iota(jnp.int32, (_BT_ROUTE, E), 1)
    sel0 = jnp.sum(jnp.where(iota == ids[:, 0:1], base_row, 0.0), axis=1,
                   keepdims=True)
    sel1 = jnp.sum(jnp.where(iota == ids[:, 1:2], base_row, 0.0), axis=1,
                   keepdims=True)
    pos0_ref[...] = (sel0 + rk[:, 0:1]).astype(jnp.int32)
    pos1_ref[...] = (sel1 + rk[:, 1:2]).astype(jnp.int32)


def _pos(ids, ranks, base3, interpret=False):
    return pl.pallas_call(
        _pos_body,
        grid=(_NB,),
        in_specs=[
            pl.BlockSpec((_BT_ROUTE, 2), lambda i: (i, 0)),
            pl.BlockSpec((_BT_ROUTE, 2), lambda i: (i, 0)),
            pl.BlockSpec((1, 1, E), lambda i: (i, 0, 0)),
        ],
        out_specs=[
            pl.BlockSpec((_BT_ROUTE, 1), lambda i: (i, 0)),
            pl.BlockSpec((_BT_ROUTE, 1), lambda i: (i, 0)),
        ],
        out_shape=[
            jax.ShapeDtypeStruct((T, 1), jnp.int32),
            jax.ShapeDtypeStruct((T, 1), jnp.int32),
        ],
        interpret=interpret,
    )(ids, ranks, base3)


def _sc_scatter_body(pos0_h, pos1_h, x_h, xs_h,
                     posv0, posv1, xbuf, sem0, sem1):
    c = lax.axis_index("c")
    s = lax.axis_index("s")
    wid = s * _NC + c
    t0 = wid * _CHT
    for q in range(_CHT // 32):
        pltpu.sync_copy(pos0_h.at[pl.ds(t0 + 32 * q, 32)], posv0.at[q])
        pltpu.sync_copy(pos1_h.at[pl.ds(t0 + 32 * q, 32)], posv1.at[q])
    for q in range(_CHT // 32):
        pltpu.sync_copy(x_h.at[pl.ds(t0 + 32 * q, 32)], xbuf)
        d0 = pltpu.async_copy(xbuf, xs_h.at[posv0.at[q]], sem0)
        d1 = pltpu.async_copy(xbuf, xs_h.at[posv1.at[q]], sem1)
        d0.wait()
        d1.wait()


def _sc_scatter(pos0, pos1, x, interpret=False):
    mesh = plsc.VectorSubcoreMesh(
        core_axis_name="c", subcore_axis_name="s", num_cores=_NC, num_subcores=_NS
    )
    fn = pl.kernel(
        _sc_scatter_body,
        out_type=jax.ShapeDtypeStruct((S, D), jnp.float32),
        mesh=mesh,
        scratch_types=[
            pltpu.VMEM((_CHT // 32, 32), jnp.int32),
            pltpu.VMEM((_CHT // 32, 32), jnp.int32),
            pltpu.VMEM((32, D), jnp.float32),
            pltpu.SemaphoreType.DMA,
            pltpu.SemaphoreType.DMA,
        ],
        interpret=interpret,
    )
    return fn(pos0, pos1, x)


def _gmm_body(se_ref, st_ref, fi_ref, lo_ref, hi_ref,
              x_ref, wg_ref, wu_ref, wd_ref, out_ref):
    i = pl.program_id(0)
    xb = x_ref[...].astype(jnp.bfloat16)
    g = jnp.dot(xb, wg_ref[0], preferred_element_type=jnp.float32)
    u = jnp.dot(xb, wu_ref[0], preferred_element_type=jnp.float32)
    h = (g * jax.nn.sigmoid(g) * u).astype(jnp.bfloat16)
    y = jnp.dot(h, wd_ref[0], preferred_element_type=jnp.float32)
    row = st_ref[i] * _TM + jax.lax.broadcasted_iota(jnp.int32, (_TM, 1), 0)
    mask = (row >= lo_ref[i]) & (row < hi_ref[i])
    contrib = jnp.where(mask, y, 0.0)

    @pl.when(fi_ref[i] == 1)
    def _():
        out_ref[...] = contrib

    @pl.when(fi_ref[i] == 0)
    def _():
        out_ref[...] = out_ref[...] + contrib


def _gmm(se, st, fi, lo, hi, x_sorted, wg, wu, wd, interpret=False):
    grid_spec = pltpu.PrefetchScalarGridSpec(
        num_scalar_prefetch=5,
        grid=(_G,),
        in_specs=[
            pl.BlockSpec((_TM, D), lambda i, se, st, fi, lo, hi: (st[i], 0)),
            pl.BlockSpec((1, D, I), lambda i, se, st, fi, lo, hi: (se[i], 0, 0)),
            pl.BlockSpec((1, D, I), lambda i, se, st, fi, lo, hi: (se[i], 0, 0)),
            pl.BlockSpec((1, I, D), lambda i, se, st, fi, lo, hi: (se[i], 0, 0)),
        ],
        out_specs=pl.BlockSpec((_TM, D), lambda i, se, st, fi, lo, hi: (st[i], 0)),
    )
    return pl.pallas_call(
        _gmm_body,
        grid_spec=grid_spec,
        out_shape=jax.ShapeDtypeStruct((S, D), jnp.float32),
        interpret=interpret,
    )(se, st, fi, lo, hi, x_sorted, wg, wu, wd)


def _sc_gather_body(ys_h, pos0_h, pos1_h, y0_h, y1_h,
                    posv0, posv1, ybuf0, ybuf1, sem0, sem1):
    c = lax.axis_index("c")
    s = lax.axis_index("s")
    wid = s * _NC + c
    t0 = wid * _CHT
    for q in range(_CHT // 32):
        pltpu.sync_copy(pos0_h.at[pl.ds(t0 + 32 * q, 32)], posv0.at[q])
        pltpu.sync_copy(pos1_h.at[pl.ds(t0 + 32 * q, 32)], posv1.at[q])
    for q in range(_CHT // 32):
        d0 = pltpu.async_copy(ys_h.at[posv0.at[q]], ybuf0, sem0)
        d1 = pltpu.async_copy(ys_h.at[posv1.at[q]], ybuf1, sem1)
        d0.wait()
        d1.wait()
        pltpu.sync_copy(ybuf0, y0_h.at[pl.ds(t0 + 32 * q, 32)])
        pltpu.sync_copy(ybuf1, y1_h.at[pl.ds(t0 + 32 * q, 32)])


def _sc_gather(y_sorted, pos0, pos1, interpret=False):
    mesh = plsc.VectorSubcoreMesh(
        core_axis_name="c", subcore_axis_name="s", num_cores=_NC, num_subcores=_NS
    )
    fn = pl.kernel(
        _sc_gather_body,
        out_type=[
            jax.ShapeDtypeStruct((T, D), jnp.float32),
            jax.ShapeDtypeStruct((T, D), jnp.float32),
        ],
        mesh=mesh,
        scratch_types=[
            pltpu.VMEM((_CHT // 32, 32), jnp.int32),
            pltpu.VMEM((_CHT // 32, 32), jnp.int32),
            pltpu.VMEM((32, D), jnp.float32),
            pltpu.VMEM((32, D), jnp.float32),
            pltpu.SemaphoreType.DMA,
            pltpu.SemaphoreType.DMA,
        ],
        interpret=interpret,
    )
    return fn(y_sorted, pos0, pos1)


def _finish_body(x_ref, wg_ref, wu_ref, wd_ref, y0_ref, y1_ref, w0_ref, w1_ref,
                 out_ref):
    x = x_ref[...]
    g = jnp.dot(x, wg_ref[...], preferred_element_type=jnp.float32)
    u = jnp.dot(x, wu_ref[...], preferred_element_type=jnp.float32)
    h = (g * jax.nn.sigmoid(g) * u).astype(jnp.bfloat16)
    y = jnp.dot(h, wd_ref[...], preferred_element_type=jnp.float32)
    out_ref[...] = y + w0_ref[...] * y0_ref[...] + w1_ref[...] * y1_ref[...]


def _finish(xb, wg, wu, wd, y0t, y1t, w0c, w1c, interpret=False):
    nt = T // _BT_FIN
    return pl.pallas_call(
        _finish_body,
        grid=(nt,),
        in_specs=[
            pl.BlockSpec((_BT_FIN, D), lambda i: (i, 0)),
            pl.BlockSpec((D, I), lambda i: (0, 0)),
            pl.BlockSpec((D, I), lambda i: (0, 0)),
            pl.BlockSpec((I, D), lambda i: (0, 0)),
            pl.BlockSpec((_BT_FIN, D), lambda i: (i, 0)),
            pl.BlockSpec((_BT_FIN, D), lambda i: (i, 0)),
            pl.BlockSpec((_BT_FIN, 1), lambda i: (i, 0)),
            pl.BlockSpec((_BT_FIN, 1), lambda i: (i, 0)),
        ],
        out_specs=pl.BlockSpec((_BT_FIN, D), lambda i: (i, 0)),
        out_shape=jax.ShapeDtypeStruct((T, D), jnp.float32),
        interpret=interpret,
    )(xb, wg, wu, wd, y0t, y1t, w0c, w1c)


def _kernel_impl(hidden_states, Wg_sh, Wu_sh, Wd_sh, W_router, e_bias,
                 We_gate, We_up, We_down, interpret=False):
    x = hidden_states
    ids, w01, cnt_blk, ranks = _routing(x, W_router, e_bias, interpret=interpret)
    base_flat, se, st, fi, lo, hi = _dispatch_meta(cnt_blk.reshape(_NB, E))

    base3 = base_flat.reshape(_NB, 1, E)
    pos0, pos1 = _pos(ids, ranks, base3, interpret=interpret)
    pos0 = pos0.reshape(T)
    pos1 = pos1.reshape(T)
    x_sorted = _sc_scatter(pos0, pos1, x, interpret=interpret)

    wg = We_gate.astype(jnp.bfloat16)
    wu = We_up.astype(jnp.bfloat16)
    wd = We_down.astype(jnp.bfloat16)
    y_sorted = _gmm(se, st, fi, lo, hi, x_sorted, wg, wu, wd, interpret=interpret)

    y0t, y1t = _sc_gather(y_sorted, pos0, pos1, interpret=interpret)

    xb = x.astype(jnp.bfloat16)
    out = _finish(xb, Wg_sh.astype(jnp.bfloat16), Wu_sh.astype(jnp.bfloat16),
                  Wd_sh.astype(jnp.bfloat16), y0t, y1t,
                  w01[:, 0:1], w01[:, 1:2], interpret=interpret)
    return out


def kernel(hidden_states, Wg_sh, Wu_sh, Wd_sh, W_router, e_bias, We_gate, We_up, We_down):
    return _kernel_impl(hidden_states, Wg_sh, Wu_sh, Wd_sh, W_router, e_bias,
                        We_gate, We_up, We_down)


# speedup vs baseline: 2.6685x; 1.1667x over previous
import functools

import jax
import jax.numpy as jnp
from jax import lax
from jax.experimental import pallas as pl
from jax.experimental.pallas import tpu as pltpu
from jax.experimental.pallas import tpu_sc as plsc

T = 4096
D = 1024
I = 512
E = 16
TOP_K = 2
N_GROUP = 4
TOPK_GROUP = 2
ROUTED_SCALING_FACTOR = 2.5

S = T * TOP_K
_BT_ROUTE = 512
_NB = T // _BT_ROUTE
_TM = 256
_NT = S // _TM
_G = _NT + E - 1
_BT_FIN = 1024

_NC = 2
_NS = 16
_NW = _NC * _NS
_CHT = T // _NW


def _lane_roll(x, s):
    return jnp.concatenate([x[:, s:], x[:, :s]], axis=1)


def _cumsum_rows_excl(x):
    inc = x
    s = 1
    while s < x.shape[0]:
        inc = inc + jnp.concatenate(
            [jnp.zeros((s, inc.shape[1]), inc.dtype), inc[:-s]], axis=0
        )
        s *= 2
    return inc - x


def _routing_body(x_ref, wr_ref, bias_ref, ids_ref, w_ref, cnt_ref, rk_ref):
    x = x_ref[...]
    logits = jnp.dot(x, wr_ref[...], preferred_element_type=jnp.float32)
    scores = jax.nn.sigmoid(logits)
    sfc = scores + bias_ref[...]

    gs_cols = []
    for g in range(N_GROUP):
        a = sfc[:, 4 * g + 0 : 4 * g + 1]
        b = sfc[:, 4 * g + 1 : 4 * g + 2]
        c = sfc[:, 4 * g + 2 : 4 * g + 3]
        d = sfc[:, 4 * g + 3 : 4 * g + 4]
        hi1, lo1 = jnp.maximum(a, b), jnp.minimum(a, b)
        hi2, lo2 = jnp.maximum(c, d), jnp.minimum(c, d)
        top1 = jnp.maximum(hi1, hi2)
        top2 = jnp.maximum(jnp.minimum(hi1, hi2), jnp.maximum(lo1, lo2))
        gs_cols.append(top1 + top2)

    grank = [jnp.zeros_like(gs_cols[0]) for _ in range(N_GROUP)]
    for g in range(N_GROUP):
        for g2 in range(N_GROUP):
            if g2 == g:
                continue
            beats = (gs_cols[g2] >= gs_cols[g]) if g2 < g else (gs_cols[g2] > gs_cols[g])
            grank[g] = grank[g] + beats.astype(jnp.float32)
    gmask = [grank[g] < TOPK_GROUP for g in range(N_GROUP)]
    ms = jnp.concatenate(
        [jnp.where(gmask[g], sfc[:, 4 * g : 4 * g + 4], 0.0) for g in range(N_GROUP)],
        axis=1,
    )

    lane = jax.lax.broadcasted_iota(jnp.int32, ms.shape, 1)
    rank = jnp.zeros_like(ms)
    for s in range(1, E):
        rolled = _lane_roll(ms, s)
        wraps = lane >= (E - s)
        beats = (rolled > ms) | ((rolled == ms) & wraps)
        rank = rank + beats.astype(jnp.float32)
    chosen = rank < TOP_K
    chf = chosen.astype(jnp.float32)

    wsum = jnp.sum(jnp.where(chosen, scores, 0.0), axis=1, keepdims=True) + 1e-20
    cw = (scores / wsum) * ROUTED_SCALING_FACTOR

    lanef = lane.astype(jnp.float32)
    id0 = jnp.min(jnp.where(chosen, lanef, 1e9), axis=1, keepdims=True)
    id1 = jnp.max(jnp.where(chosen, lanef, -1.0), axis=1, keepdims=True)
    is0 = lanef == id0
    is1 = lanef == id1
    w0 = jnp.sum(jnp.where(is0, cw, 0.0), axis=1, keepdims=True)
    w1 = jnp.sum(jnp.where(is1, cw, 0.0), axis=1, keepdims=True)

    cum = _cumsum_rows_excl(chf)
    r0 = jnp.sum(jnp.where(is0, cum, 0.0), axis=1, keepdims=True)
    r1 = jnp.sum(jnp.where(is1, cum, 0.0), axis=1, keepdims=True)

    ids_ref[...] = jnp.concatenate([id0, id1], axis=1).astype(jnp.int32)
    w_ref[...] = jnp.concatenate([w0, w1], axis=1)
    cnt_ref[...] = jnp.sum(chf, axis=0, keepdims=True).astype(jnp.int32)[None]
    rk_ref[...] = jnp.concatenate([r0, r1], axis=1).astype(jnp.int32)


def _routing(x, w_router, e_bias, interpret=False):
    return pl.pallas_call(
        _routing_body,
        grid=(_NB,),
        in_specs=[
            pl.BlockSpec((_BT_ROUTE, D), lambda i: (i, 0)),
            pl.BlockSpec((D, E), lambda i: (0, 0)),
            pl.BlockSpec((1, E), lambda i: (0, 0)),
        ],
        out_specs=[
            pl.BlockSpec((_BT_ROUTE, 2), lambda i: (i, 0)),
            pl.BlockSpec((_BT_ROUTE, 2), lambda i: (i, 0)),
            pl.BlockSpec((1, 1, E), lambda i: (i, 0, 0)),
            pl.BlockSpec((_BT_ROUTE, 2), lambda i: (i, 0)),
        ],
        out_shape=[
            jax.ShapeDtypeStruct((T, 2), jnp.int32),
            jax.ShapeDtypeStruct((T, 2), jnp.float32),
            jax.ShapeDtypeStruct((_NB, 1, E), jnp.int32),
            jax.ShapeDtypeStruct((T, 2), jnp.int32),
        ],
        interpret=interpret,
    )(x, w_router, e_bias.reshape(1, E))


def _dispatch_meta(cnt_blk):
    tot = jnp.sum(cnt_blk, axis=0)
    offs = jnp.concatenate([jnp.zeros((1,), jnp.int32), jnp.cumsum(tot)]).astype(jnp.int32)
    blk_excl = jnp.cumsum(cnt_blk, axis=0) - cnt_blk
    base_flat = (offs[:E][None, :] + blk_excl).reshape(_NB * E).astype(jnp.int32)

    first_tile = offs[:E] // _TM
    ntiles = jnp.where(tot > 0, -(-offs[1:] // _TM) - first_tile, 0).astype(jnp.int32)
    start = (jnp.cumsum(ntiles) - ntiles).astype(jnp.int32)
    total = jnp.sum(ntiles)
    i = jnp.arange(_G, dtype=jnp.int32)
    se = jnp.sum((start[None, :] + ntiles[None, :]) <= i[:, None], axis=1).astype(jnp.int32)
    se = jnp.clip(se, 0, E - 1)
    st = jnp.take(first_tile, se) + (i - jnp.take(start, se))
    valid = i < total
    st = jnp.where(valid, st, _NT - 1).astype(jnp.int32)
    lo = jnp.where(valid, jnp.maximum(jnp.take(offs, se), st * _TM), 0).astype(jnp.int32)
    hi = jnp.where(valid, jnp.minimum(jnp.take(offs, se + 1), (st + 1) * _TM), 0).astype(jnp.int32)
    fi = jnp.concatenate(
        [jnp.ones((1,), jnp.int32), (st[1:] != st[:-1]).astype(jnp.int32)]
    )
    return base_flat, se, st, fi, lo, hi


def _pos_body(ids_ref, rk_ref, base_ref, pos0_ref, pos1_ref):
    ids = ids_ref[...]
    rk = rk_ref[...].astype(jnp.float32)
    base_row = base_ref[0].astype(jnp.float32)
    iota = jax.lax.broadcasted_iota(jnp.int32, (_BT_ROUTE, E), 1)
    sel0 = jnp.sum(jnp.where(iota == ids[:, 0:1], base_row, 0.0), axis=1,
                   keepdims=True)
    sel1 = jnp.sum(jnp.where(iota == ids[:, 1:2], base_row, 0.0), axis=1,
                   keepdims=True)
    pos0_ref[...] = (sel0 + rk[:, 0:1]).astype(jnp.int32)
    pos1_ref[...] = (sel1 + rk[:, 1:2]).astype(jnp.int32)


def _pos(ids, ranks, base3, interpret=False):
    return pl.pallas_call(
        _pos_body,
        grid=(_NB,),
        in_specs=[
            pl.BlockSpec((_BT_ROUTE, 2), lambda i: (i, 0)),
            pl.BlockSpec((_BT_ROUTE, 2), lambda i: (i, 0)),
            pl.BlockSpec((1, 1, E), lambda i: (i, 0, 0)),
        ],
        out_specs=[
            pl.BlockSpec((_BT_ROUTE, 1), lambda i: (i, 0)),
            pl.BlockSpec((_BT_ROUTE, 1), lambda i: (i, 0)),
        ],
        out_shape=[
            jax.ShapeDtypeStruct((T, 1), jnp.int32),
            jax.ShapeDtypeStruct((T, 1), jnp.int32),
        ],
        interpret=interpret,
    )(ids, ranks, base3)


_CH = 16
_NQ = _CHT // _CH


def _sc_scatter_body(pos0_h, pos1_h, x_h, xs_h,
                     posv0, posv1, xb0, xb1,
                     seml0, seml1, sem00, sem01, sem10, sem11):
    c = lax.axis_index("c")
    s = lax.axis_index("s")
    wid = s * _NC + c
    t0 = wid * _CHT
    pltpu.sync_copy(pos0_h.at[pl.ds(wid * _NQ, _NQ)], posv0)
    pltpu.sync_copy(pos1_h.at[pl.ds(wid * _NQ, _NQ)], posv1)
    xb = (xb0, xb1)
    seml = (seml0, seml1)
    sems = ((sem00, sem01), (sem10, sem11))
    lds = [None, None]
    pend = [None, None]
    lds[0] = pltpu.async_copy(x_h.at[pl.ds(t0, _CH)], xb0, seml0)
    for q in range(_NQ):
        b = q % 2
        lds[b].wait()
        if q + 1 < _NQ:
            nb = 1 - b
            if pend[nb] is not None:
                pend[nb][0].wait()
                pend[nb][1].wait()
                pend[nb] = None
            lds[nb] = pltpu.async_copy(
                x_h.at[pl.ds(t0 + _CH * (q + 1), _CH)], xb[nb], seml[nb])
        d0 = pltpu.async_copy(xb[b], xs_h.at[posv0.at[q]], sems[b][0])
        d1 = pltpu.async_copy(xb[b], xs_h.at[posv1.at[q]], sems[b][1])
        pend[b] = (d0, d1)
    for p in pend:
        if p is not None:
            p[0].wait()
            p[1].wait()


def _sc_scatter(pos0r, pos1r, x, interpret=False):
    mesh = plsc.VectorSubcoreMesh(
        core_axis_name="c", subcore_axis_name="s", num_cores=_NC, num_subcores=_NS
    )
    fn = pl.kernel(
        _sc_scatter_body,
        out_type=jax.ShapeDtypeStruct((S, D), jnp.float32),
        mesh=mesh,
        scratch_types=[
            pltpu.VMEM((_NQ, _CH), jnp.int32),
            pltpu.VMEM((_NQ, _CH), jnp.int32),
            pltpu.VMEM((_CH, D), jnp.float32),
            pltpu.VMEM((_CH, D), jnp.float32),
        ] + [pltpu.SemaphoreType.DMA] * 6,
        interpret=interpret,
    )
    return fn(pos0r, pos1r, x)


def _gmm_body(se_ref, st_ref, fi_ref, lo_ref, hi_ref,
              x_ref, wg_ref, wu_ref, wd_ref, out_ref):
    i = pl.program_id(0)
    xb = x_ref[...]
    g = jnp.dot(xb, wg_ref[0], preferred_element_type=jnp.float32)
    u = jnp.dot(xb, wu_ref[0], preferred_element_type=jnp.float32)
    h = g * jax.nn.sigmoid(g) * u
    y = jnp.dot(h, wd_ref[0], preferred_element_type=jnp.float32)
    row = st_ref[i] * _TM + jax.lax.broadcasted_iota(jnp.int32, (_TM, 1), 0)
    mask = (row >= lo_ref[i]) & (row < hi_ref[i])
    contrib = jnp.where(mask, y, 0.0)

    @pl.when(fi_ref[i] == 1)
    def _():
        out_ref[...] = contrib

    @pl.when(fi_ref[i] == 0)
    def _():
        out_ref[...] = out_ref[...] + contrib


def _gmm(se, st, fi, lo, hi, x_sorted, wg, wu, wd, interpret=False):
    grid_spec = pltpu.PrefetchScalarGridSpec(
        num_scalar_prefetch=5,
        grid=(_G,),
        in_specs=[
            pl.BlockSpec((_TM, D), lambda i, se, st, fi, lo, hi: (st[i], 0)),
            pl.BlockSpec((1, D, I), lambda i, se, st, fi, lo, hi: (se[i], 0, 0)),
            pl.BlockSpec((1, D, I), lambda i, se, st, fi, lo, hi: (se[i], 0, 0)),
            pl.BlockSpec((1, I, D), lambda i, se, st, fi, lo, hi: (se[i], 0, 0)),
        ],
        out_specs=pl.BlockSpec((_TM, D), lambda i, se, st, fi, lo, hi: (st[i], 0)),
    )
    return pl.pallas_call(
        _gmm_body,
        grid_spec=grid_spec,
        out_shape=jax.ShapeDtypeStruct((S, D), jnp.float32),
        interpret=interpret,
    )(se, st, fi, lo, hi, x_sorted, wg, wu, wd)


def _sc_gather_body(ys_h, pos0_h, pos1_h, y0_h, y1_h,
                    posv0, posv1, yb00, yb01, yb10, yb11,
                    semg00, semg01, semg10, semg11,
                    semw00, semw01, semw10, semw11):
    c = lax.axis_index("c")
    s = lax.axis_index("s")
    wid = s * _NC + c
    t0 = wid * _CHT
    pltpu.sync_copy(pos0_h.at[pl.ds(wid * _NQ, _NQ)], posv0)
    pltpu.sync_copy(pos1_h.at[pl.ds(wid * _NQ, _NQ)], posv1)
    posv = (posv0, posv1)
    yb = ((yb00, yb01), (yb10, yb11))
    semg = ((semg00, semg01), (semg10, semg11))
    semw = ((semw00, semw01), (semw10, semw11))
    yh = (y0_h, y1_h)
    gd = [[None, None], [None, None]]
    wd = [[None, None], [None, None]]
    for st in range(2):
        gd[st][0] = pltpu.async_copy(ys_h.at[posv[st].at[0]], yb[st][0],
                                     semg[st][0])
    for q in range(_NQ):
        b = q % 2
        for st in range(2):
            gd[st][b].wait()
        if q + 1 < _NQ:
            nb = 1 - b
            for st in range(2):
                if wd[st][nb] is not None:
                    wd[st][nb].wait()
                    wd[st][nb] = None
                gd[st][nb] = pltpu.async_copy(
                    ys_h.at[posv[st].at[q + 1]], yb[st][nb], semg[st][nb])
        for st in range(2):
            wd[st][b] = pltpu.async_copy(
                yb[st][b], yh[st].at[pl.ds(t0 + _CH * q, _CH)], semw[st][b])
    for st in range(2):
        for b in range(2):
            if wd[st][b] is not None:
                wd[st][b].wait()


def _sc_gather(y_sorted, pos0r, pos1r, interpret=False):
    mesh = plsc.VectorSubcoreMesh(
        core_axis_name="c", subcore_axis_name="s", num_cores=_NC, num_subcores=_NS
    )
    fn = pl.kernel(
        _sc_gather_body,
        out_type=[
            jax.ShapeDtypeStruct((T, D), jnp.float32),
            jax.ShapeDtypeStruct((T, D), jnp.float32),
        ],
        mesh=mesh,
        scratch_types=[
            pltpu.VMEM((_NQ, _CH), jnp.int32),
            pltpu.VMEM((_NQ, _CH), jnp.int32),
            pltpu.VMEM((_CH, D), jnp.float32),
            pltpu.VMEM((_CH, D), jnp.float32),
            pltpu.VMEM((_CH, D), jnp.float32),
            pltpu.VMEM((_CH, D), jnp.float32),
        ] + [pltpu.SemaphoreType.DMA] * 8,
        interpret=interpret,
    )
    return fn(y_sorted, pos0r, pos1r)


def _finish_body(x_ref, wg_ref, wu_ref, wd_ref, y0_ref, y1_ref, w0_ref, w1_ref,
                 out_ref):
    x = x_ref[...]
    g = jnp.dot(x, wg_ref[...], preferred_element_type=jnp.float32)
    u = jnp.dot(x, wu_ref[...], preferred_element_type=jnp.float32)
    h = g * jax.nn.sigmoid(g) * u
    y = jnp.dot(h, wd_ref[...], preferred_element_type=jnp.float32)
    out_ref[...] = y + w0_ref[...] * y0_ref[...] + w1_ref[...] * y1_ref[...]


def _finish(xb, wg, wu, wd, y0t, y1t, w0c, w1c, interpret=False):
    nt = T // _BT_FIN
    return pl.pallas_call(
        _finish_body,
        grid=(nt,),
        in_specs=[
            pl.BlockSpec((_BT_FIN, D), lambda i: (i, 0)),
            pl.BlockSpec((D, I), lambda i: (0, 0)),
            pl.BlockSpec((D, I), lambda i: (0, 0)),
            pl.BlockSpec((I, D), lambda i: (0, 0)),
            pl.BlockSpec((_BT_FIN, D), lambda i: (i, 0)),
            pl.BlockSpec((_BT_FIN, D), lambda i: (i, 0)),
            pl.BlockSpec((_BT_FIN, 1), lambda i: (i, 0)),
            pl.BlockSpec((_BT_FIN, 1), lambda i: (i, 0)),
        ],
        out_specs=pl.BlockSpec((_BT_FIN, D), lambda i: (i, 0)),
        out_shape=jax.ShapeDtypeStruct((T, D), jnp.float32),
        interpret=interpret,
    )(xb, wg, wu, wd, y0t, y1t, w0c, w1c)


def _kernel_impl(hidden_states, Wg_sh, Wu_sh, Wd_sh, W_router, e_bias,
                 We_gate, We_up, We_down, interpret=False):
    x = hidden_states
    ids, w01, cnt_blk, ranks = _routing(x, W_router, e_bias, interpret=interpret)
    base_flat, se, st, fi, lo, hi = _dispatch_meta(cnt_blk.reshape(_NB, E))

    base3 = base_flat.reshape(_NB, 1, E)
    pos0, pos1 = _pos(ids, ranks, base3, interpret=interpret)
    pos0r = pos0.reshape(T // _CH, _CH)
    pos1r = pos1.reshape(T // _CH, _CH)
    x_sorted = _sc_scatter(pos0r, pos1r, x, interpret=interpret)

    y_sorted = _gmm(se, st, fi, lo, hi, x_sorted, We_gate, We_up, We_down,
                    interpret=interpret)

    y0t, y1t = _sc_gather(y_sorted, pos0r, pos1r, interpret=interpret)

    out = _finish(x, Wg_sh, Wu_sh, Wd_sh, y0t, y1t,
                  w01[:, 0:1], w01[:, 1:2], interpret=interpret)
    return out


def kernel(hidden_states, Wg_sh, Wu_sh, Wd_sh, W_router, e_bias, We_gate, We_up, We_down):
    return _kernel_impl(hidden_states, Wg_sh, Wu_sh, Wd_sh, W_router, e_bias,
                        We_gate, We_up, We_down)


# speedup vs baseline: 3.2468x; 1.2167x over previous
import functools

import jax
import jax.numpy as jnp
from jax import lax
from jax.experimental import pallas as pl
from jax.experimental.pallas import tpu as pltpu
from jax.experimental.pallas import tpu_sc as plsc

T = 4096
D = 1024
I = 512
E = 16
TOP_K = 2
N_GROUP = 4
TOPK_GROUP = 2
ROUTED_SCALING_FACTOR = 2.5

S = T * TOP_K
_BT_ROUTE = 512
_NB = T // _BT_ROUTE
_TM = 256
_NT = S // _TM
_G = _NT + E - 1
_BT_FIN = 1024

_NC = 2
_NS = 16
_NW = _NC * _NS
_CHT = T // _NW


_MROWS = 8
_MCOLS = 64


def _row_roll(x, s):
    return jnp.concatenate([x[s:], x[:s]], axis=0)


def _route2_body(x_ref, wrt_ref, bias_ref, pos0_ref, pos1_ref, w0_ref, w1_ref,
                 meta_ref):
    logits = jax.lax.dot_general(
        wrt_ref[...], x_ref[...], (((1,), (1,)), ((), ())),
        preferred_element_type=jnp.float32)
    scores = jax.nn.sigmoid(logits)
    sfc = scores + bias_ref[...]

    gs_rows = []
    for g in range(N_GROUP):
        a = sfc[4 * g + 0 : 4 * g + 1]
        b = sfc[4 * g + 1 : 4 * g + 2]
        c = sfc[4 * g + 2 : 4 * g + 3]
        d = sfc[4 * g + 3 : 4 * g + 4]
        hi1, lo1 = jnp.maximum(a, b), jnp.minimum(a, b)
        hi2, lo2 = jnp.maximum(c, d), jnp.minimum(c, d)
        top1 = jnp.maximum(hi1, hi2)
        top2 = jnp.maximum(jnp.minimum(hi1, hi2), jnp.maximum(lo1, lo2))
        gs_rows.append(top1 + top2)

    grank = [jnp.zeros_like(gs_rows[0]) for _ in range(N_GROUP)]
    for g in range(N_GROUP):
        for g2 in range(N_GROUP):
            if g2 == g:
                continue
            beats = (gs_rows[g2] >= gs_rows[g]) if g2 < g else (gs_rows[g2] > gs_rows[g])
            grank[g] = grank[g] + beats.astype(jnp.float32)
    gmask = [grank[g] < TOPK_GROUP for g in range(N_GROUP)]
    ms = jnp.concatenate(
        [jnp.where(gmask[g], sfc[4 * g : 4 * g + 4], 0.0) for g in range(N_GROUP)],
        axis=0,
    )

    rowi = jax.lax.broadcasted_iota(jnp.int32, ms.shape, 0)
    rank = jnp.zeros_like(ms)
    for s in range(1, E):
        rolled = _row_roll(ms, s)
        wraps = rowi >= (E - s)
        beats = (rolled > ms) | ((rolled == ms) & wraps)
        rank = rank + beats.astype(jnp.float32)
    chosen = rank < TOP_K
    chf = chosen.astype(jnp.float32)

    wsum = jnp.sum(jnp.where(chosen, scores, 0.0), axis=0, keepdims=True) + 1e-20
    cw = (scores / wsum) * ROUTED_SCALING_FACTOR

    rowf = rowi.astype(jnp.float32)
    id0 = jnp.min(jnp.where(chosen, rowf, 1e9), axis=0, keepdims=True)
    id1 = jnp.max(jnp.where(chosen, rowf, -1.0), axis=0, keepdims=True)
    is0 = rowf == id0
    is1 = rowf == id1
    w0_ref[...] = jnp.sum(jnp.where(is0, cw, 0.0), axis=0, keepdims=True)
    w1_ref[...] = jnp.sum(jnp.where(is1, cw, 0.0), axis=0, keepdims=True)

    cnt = jnp.sum(chf, axis=1, keepdims=True)
    csum = cnt
    s = 1
    while s < E:
        csum = csum + jnp.concatenate(
            [jnp.zeros((s, 1), jnp.float32), csum[:-s]], axis=0)
        s *= 2
    offs = csum - cnt

    cum = chf
    s = 1
    while s < T:
        cum = cum + jnp.concatenate(
            [jnp.zeros((E, s), jnp.float32), cum[:, :-s]], axis=1)
        s *= 2
    cum = cum - chf

    posm = offs + cum
    pos0_ref[...] = jnp.sum(jnp.where(is0, posm, 0.0), axis=0,
                            keepdims=True).astype(jnp.int32)
    pos1_ref[...] = jnp.sum(jnp.where(is1, posm, 0.0), axis=0,
                            keepdims=True).astype(jnp.int32)

    offs_i = offs.astype(jnp.int32)
    cnt_i = cnt.astype(jnp.int32)
    ends_i = offs_i + cnt_i
    first_tile = jax.lax.shift_right_logical(offs_i, 8)
    ceil_end = jax.lax.shift_right_logical(ends_i + (_TM - 1), 8)
    ntiles = jnp.where(cnt_i > 0, ceil_end - first_tile, 0)
    nt_c = ntiles
    s = 1
    while s < E:
        nt_c = nt_c + jnp.concatenate(
            [jnp.zeros((s, 1), jnp.int32), nt_c[:-s]], axis=0)
        s *= 2
    startv = nt_c - ntiles
    total = jnp.sum(ntiles, axis=0, keepdims=True)

    iv = jax.lax.broadcasted_iota(jnp.int32, (1, _MCOLS), 1)
    done = startv + ntiles
    se = jnp.sum((done <= iv).astype(jnp.int32), axis=0, keepdims=True)
    se = jnp.minimum(se, E - 1)
    rowi16 = jax.lax.broadcasted_iota(jnp.int32, (E, _MCOLS), 0)
    sel = rowi16 == se

    def _pick(col):
        return jnp.sum(jnp.where(sel, col, 0), axis=0, keepdims=True)

    st = _pick(first_tile) + (iv - _pick(startv))
    valid = iv < total
    st = jnp.where(valid, st, _NT - 1)
    lo = jnp.where(valid,
                   jnp.maximum(_pick(offs_i), jax.lax.shift_left(st, 8)), 0)
    hi = jnp.where(valid,
                   jnp.minimum(_pick(ends_i), jax.lax.shift_left(st + 1, 8)), 0)
    stprev = jnp.concatenate([st[:, :1] - 1, st[:, :-1]], axis=1)
    fi = (st != stprev).astype(jnp.int32)
    zero = jnp.zeros((1, _MCOLS), jnp.int32)
    meta_ref[...] = jnp.concatenate([se, st, fi, lo, hi, zero, zero, zero],
                                    axis=0)


def _route2(x, w_router, e_bias, interpret=False):
    return pl.pallas_call(
        _route2_body,
        grid=(1,),
        in_specs=[
            pl.BlockSpec((T, D), lambda i: (0, 0)),
            pl.BlockSpec((E, D), lambda i: (0, 0)),
            pl.BlockSpec((E, 1), lambda i: (0, 0)),
        ],
        out_specs=[
            pl.BlockSpec((1, T), lambda i: (0, 0)),
            pl.BlockSpec((1, T), lambda i: (0, 0)),
            pl.BlockSpec((1, T), lambda i: (0, 0)),
            pl.BlockSpec((1, T), lambda i: (0, 0)),
            pl.BlockSpec((_MROWS, _MCOLS), lambda i: (0, 0)),
        ],
        out_shape=[
            jax.ShapeDtypeStruct((1, T), jnp.int32),
            jax.ShapeDtypeStruct((1, T), jnp.int32),
            jax.ShapeDtypeStruct((1, T), jnp.float32),
            jax.ShapeDtypeStruct((1, T), jnp.float32),
            jax.ShapeDtypeStruct((_MROWS, _MCOLS), jnp.int32),
        ],
        interpret=interpret,
    )(x, w_router.T, e_bias.reshape(E, 1))


def _lane_roll(x, s):
    return jnp.concatenate([x[:, s:], x[:, :s]], axis=1)


def _cumsum_rows_excl(x):
    inc = x
    s = 1
    while s < x.shape[0]:
        inc = inc + jnp.concatenate(
            [jnp.zeros((s, inc.shape[1]), inc.dtype), inc[:-s]], axis=0
        )
        s *= 2
    return inc - x


def _routing_body(x_ref, wr_ref, bias_ref, ids_ref, w_ref, cnt_ref, rk_ref):
    x = x_ref[...]
    logits = jnp.dot(x, wr_ref[...], preferred_element_type=jnp.float32)
    scores = jax.nn.sigmoid(logits)
    sfc = scores + bias_ref[...]

    gs_cols = []
    for g in range(N_GROUP):
        a = sfc[:, 4 * g + 0 : 4 * g + 1]
        b = sfc[:, 4 * g + 1 : 4 * g + 2]
        c = sfc[:, 4 * g + 2 : 4 * g + 3]
        d = sfc[:, 4 * g + 3 : 4 * g + 4]
        hi1, lo1 = jnp.maximum(a, b), jnp.minimum(a, b)
        hi2, lo2 = jnp.maximum(c, d), jnp.minimum(c, d)
        top1 = jnp.maximum(hi1, hi2)
        top2 = jnp.maximum(jnp.minimum(hi1, hi2), jnp.maximum(lo1, lo2))
        gs_cols.append(top1 + top2)

    grank = [jnp.zeros_like(gs_cols[0]) for _ in range(N_GROUP)]
    for g in range(N_GROUP):
        for g2 in range(N_GROUP):
            if g2 == g:
                continue
            beats = (gs_cols[g2] >= gs_cols[g]) if g2 < g else (gs_cols[g2] > gs_cols[g])
            grank[g] = grank[g] + beats.astype(jnp.float32)
    gmask = [grank[g] < TOPK_GROUP for g in range(N_GROUP)]
    ms = jnp.concatenate(
        [jnp.where(gmask[g], sfc[:, 4 * g : 4 * g + 4], 0.0) for g in range(N_GROUP)],
        axis=1,
    )

    lane = jax.lax.broadcasted_iota(jnp.int32, ms.shape, 1)
    rank = jnp.zeros_like(ms)
    for s in range(1, E):
        rolled = _lane_roll(ms, s)
        wraps = lane >= (E - s)
        beats = (rolled > ms) | ((rolled == ms) & wraps)
        rank = rank + beats.astype(jnp.float32)
    chosen = rank < TOP_K
    chf = chosen.astype(jnp.float32)

    wsum = jnp.sum(jnp.where(chosen, scores, 0.0), axis=1, keepdims=True) + 1e-20
    cw = (scores / wsum) * ROUTED_SCALING_FACTOR

    lanef = lane.astype(jnp.float32)
    id0 = jnp.min(jnp.where(chosen, lanef, 1e9), axis=1, keepdims=True)
    id1 = jnp.max(jnp.where(chosen, lanef, -1.0), axis=1, keepdims=True)
    is0 = lanef == id0
    is1 = lanef == id1
    w0 = jnp.sum(jnp.where(is0, cw, 0.0), axis=1, keepdims=True)
    w1 = jnp.sum(jnp.where(is1, cw, 0.0), axis=1, keepdims=True)

    cum = _cumsum_rows_excl(chf)
    r0 = jnp.sum(jnp.where(is0, cum, 0.0), axis=1, keepdims=True)
    r1 = jnp.sum(jnp.where(is1, cum, 0.0), axis=1, keepdims=True)

    ids_ref[...] = jnp.concatenate([id0, id1], axis=1).astype(jnp.int32)
    w_ref[...] = jnp.concatenate([w0, w1], axis=1)
    cnt_ref[...] = jnp.sum(chf, axis=0, keepdims=True).astype(jnp.int32)[None]
    rk_ref[...] = jnp.concatenate([r0, r1], axis=1).astype(jnp.int32)


def _routing(x, w_router, e_bias, interpret=False):
    return pl.pallas_call(
        _routing_body,
        grid=(_NB,),
        in_specs=[
            pl.BlockSpec((_BT_ROUTE, D), lambda i: (i, 0)),
            pl.BlockSpec((D, E), lambda i: (0, 0)),
            pl.BlockSpec((1, E), lambda i: (0, 0)),
        ],
        out_specs=[
            pl.BlockSpec((_BT_ROUTE, 2), lambda i: (i, 0)),
            pl.BlockSpec((_BT_ROUTE, 2), lambda i: (i, 0)),
            pl.BlockSpec((1, 1, E), lambda i: (i, 0, 0)),
            pl.BlockSpec((_BT_ROUTE, 2), lambda i: (i, 0)),
        ],
        out_shape=[
            jax.ShapeDtypeStruct((T, 2), jnp.int32),
            jax.ShapeDtypeStruct((T, 2), jnp.float32),
            jax.ShapeDtypeStruct((_NB, 1, E), jnp.int32),
            jax.ShapeDtypeStruct((T, 2), jnp.int32),
        ],
        interpret=interpret,
    )(x, w_router, e_bias.reshape(1, E))


def _dispatch_meta(cnt_blk):
    tot = jnp.sum(cnt_blk, axis=0)
    offs = jnp.concatenate([jnp.zeros((1,), jnp.int32), jnp.cumsum(tot)]).astype(jnp.int32)
    blk_excl = jnp.cumsum(cnt_blk, axis=0) - cnt_blk
    base_flat = (offs[:E][None, :] + blk_excl).reshape(_NB * E).astype(jnp.int32)

    first_tile = offs[:E] // _TM
    ntiles = jnp.where(tot > 0, -(-offs[1:] // _TM) - first_tile, 0).astype(jnp.int32)
    start = (jnp.cumsum(ntiles) - ntiles).astype(jnp.int32)
    total = jnp.sum(ntiles)
    i = jnp.arange(_G, dtype=jnp.int32)
    se = jnp.sum((start[None, :] + ntiles[None, :]) <= i[:, None], axis=1).astype(jnp.int32)
    se = jnp.clip(se, 0, E - 1)
    st = jnp.take(first_tile, se) + (i - jnp.take(start, se))
    valid = i < total
    st = jnp.where(valid, st, _NT - 1).astype(jnp.int32)
    lo = jnp.where(valid, jnp.maximum(jnp.take(offs, se), st * _TM), 0).astype(jnp.int32)
    hi = jnp.where(valid, jnp.minimum(jnp.take(offs, se + 1), (st + 1) * _TM), 0).astype(jnp.int32)
    fi = jnp.concatenate(
        [jnp.ones((1,), jnp.int32), (st[1:] != st[:-1]).astype(jnp.int32)]
    )
    return base_flat, se, st, fi, lo, hi


def _pos_body(ids_ref, rk_ref, base_ref, pos0_ref, pos1_ref):
    ids = ids_ref[...]
    rk = rk_ref[...].astype(jnp.float32)
    base_row = base_ref[0].astype(jnp.float32)
    iota = jax.lax.broadcasted_iota(jnp.int32, (_BT_ROUTE, E), 1)
    sel0 = jnp.sum(jnp.where(iota == ids[:, 0:1], base_row, 0.0), axis=1,
                   keepdims=True)
    sel1 = jnp.sum(jnp.where(iota == ids[:, 1:2], base_row, 0.0), axis=1,
                   keepdims=True)
    pos0_ref[...] = (sel0 + rk[:, 0:1]).astype(jnp.int32)
    pos1_ref[...] = (sel1 + rk[:, 1:2]).astype(jnp.int32)


def _pos(ids, ranks, base3, interpret=False):
    return pl.pallas_call(
        _pos_body,
        grid=(_NB,),
        in_specs=[
            pl.BlockSpec((_BT_ROUTE, 2), lambda i: (i, 0)),
            pl.BlockSpec((_BT_ROUTE, 2), lambda i: (i, 0)),
            pl.BlockSpec((1, 1, E), lambda i: (i, 0, 0)),
        ],
        out_specs=[
            pl.BlockSpec((_BT_ROUTE, 1), lambda i: (i, 0)),
            pl.BlockSpec((_BT_ROUTE, 1), lambda i: (i, 0)),
        ],
        out_shape=[
            jax.ShapeDtypeStruct((T, 1), jnp.int32),
            jax.ShapeDtypeStruct((T, 1), jnp.int32),
        ],
        interpret=interpret,
    )(ids, ranks, base3)


_CH = 16
_NQ = _CHT // _CH


def _sc_scatter_body(pos0_h, pos1_h, x_h, xs_h,
                     posv0, posv1, xb0, xb1,
                     seml0, seml1, sem00, sem01, sem10, sem11):
    c = lax.axis_index("c")
    s = lax.axis_index("s")
    wid = s * _NC + c
    t0 = wid * _CHT
    pltpu.sync_copy(pos0_h.at[pl.ds(wid * _NQ, _NQ)], posv0)
    pltpu.sync_copy(pos1_h.at[pl.ds(wid * _NQ, _NQ)], posv1)
    xb = (xb0, xb1)
    seml = (seml0, seml1)
    sems = ((sem00, sem01), (sem10, sem11))
    lds = [None, None]
    pend = [None, None]
    lds[0] = pltpu.async_copy(x_h.at[pl.ds(t0, _CH)], xb0, seml0)
    for q in range(_NQ):
        b = q % 2
        lds[b].wait()
        if q + 1 < _NQ:
            nb = 1 - b
            if pend[nb] is not None:
                pend[nb][0].wait()
                pend[nb][1].wait()
                pend[nb] = None
            lds[nb] = pltpu.async_copy(
                x_h.at[pl.ds(t0 + _CH * (q + 1), _CH)], xb[nb], seml[nb])
        d0 = pltpu.async_copy(xb[b], xs_h.at[posv0.at[q]], sems[b][0])
        d1 = pltpu.async_copy(xb[b], xs_h.at[posv1.at[q]], sems[b][1])
        pend[b] = (d0, d1)
    for p in pend:
        if p is not None:
            p[0].wait()
            p[1].wait()


def _sc_scatter(pos0r, pos1r, x, interpret=False):
    mesh = plsc.VectorSubcoreMesh(
        core_axis_name="c", subcore_axis_name="s", num_cores=_NC, num_subcores=_NS
    )
    fn = pl.kernel(
        _sc_scatter_body,
        out_type=jax.ShapeDtypeStruct((S, D), jnp.float32),
        mesh=mesh,
        scratch_types=[
            pltpu.VMEM((_NQ, _CH), jnp.int32),
            pltpu.VMEM((_NQ, _CH), jnp.int32),
            pltpu.VMEM((_CH, D), jnp.float32),
            pltpu.VMEM((_CH, D), jnp.float32),
        ] + [pltpu.SemaphoreType.DMA] * 6,
        interpret=interpret,
    )
    return fn(pos0r, pos1r, x)


def _gmm_body(meta_ref, x_ref, wg_ref, wu_ref, wd_ref, out_ref):
    i = pl.program_id(0)
    xb = x_ref[...]
    g = jnp.dot(xb, wg_ref[0], preferred_element_type=jnp.float32)
    u = jnp.dot(xb, wu_ref[0], preferred_element_type=jnp.float32)
    h = g * jax.nn.sigmoid(g) * u
    y = jnp.dot(h, wd_ref[0], preferred_element_type=jnp.float32)
    row = meta_ref[1, i] * _TM + jax.lax.broadcasted_iota(jnp.int32, (_TM, 1), 0)
    mask = (row >= meta_ref[3, i]) & (row < meta_ref[4, i])
    contrib = jnp.where(mask, y, 0.0)

    @pl.when(meta_ref[2, i] == 1)
    def _():
        out_ref[...] = contrib

    @pl.when(meta_ref[2, i] == 0)
    def _():
        out_ref[...] = out_ref[...] + contrib


def _gmm(meta, x_sorted, wg, wu, wd, interpret=False):
    grid_spec = pltpu.PrefetchScalarGridSpec(
        num_scalar_prefetch=1,
        grid=(_G,),
        in_specs=[
            pl.BlockSpec((_TM, D), lambda i, m: (m[1, i], 0)),
            pl.BlockSpec((1, D, I), lambda i, m: (m[0, i], 0, 0)),
            pl.BlockSpec((1, D, I), lambda i, m: (m[0, i], 0, 0)),
            pl.BlockSpec((1, I, D), lambda i, m: (m[0, i], 0, 0)),
        ],
        out_specs=pl.BlockSpec((_TM, D), lambda i, m: (m[1, i], 0)),
    )
    return pl.pallas_call(
        _gmm_body,
        grid_spec=grid_spec,
        out_shape=jax.ShapeDtypeStruct((S, D), jnp.float32),
        interpret=interpret,
    )(meta, x_sorted, wg, wu, wd)


def _sc_gather_body(ys_h, pos0_h, pos1_h, y0_h, y1_h,
                    posv0, posv1, yb00, yb01, yb10, yb11,
                    semg00, semg01, semg10, semg11,
                    semw00, semw01, semw10, semw11):
    c = lax.axis_index("c")
    s = lax.axis_index("s")
    wid = s * _NC + c
    t0 = wid * _CHT
    pltpu.sync_copy(pos0_h.at[pl.ds(wid * _NQ, _NQ)], posv0)
    pltpu.sync_copy(pos1_h.at[pl.ds(wid * _NQ, _NQ)], posv1)
    posv = (posv0, posv1)
    yb = ((yb00, yb01), (yb10, yb11))
    semg = ((semg00, semg01), (semg10, semg11))
    semw = ((semw00, semw01), (semw10, semw11))
    yh = (y0_h, y1_h)
    gd = [[None, None], [None, None]]
    wd = [[None, None], [None, None]]
    for st in range(2):
        gd[st][0] = pltpu.async_copy(ys_h.at[posv[st].at[0]], yb[st][0],
                                     semg[st][0])
    for q in range(_NQ):
        b = q % 2
        for st in range(2):
            gd[st][b].wait()
        if q + 1 < _NQ:
            nb = 1 - b
            for st in range(2):
                if wd[st][nb] is not None:
                    wd[st][nb].wait()
                    wd[st][nb] = None
                gd[st][nb] = pltpu.async_copy(
                    ys_h.at[posv[st].at[q + 1]], yb[st][nb], semg[st][nb])
        for st in range(2):
            wd[st][b] = pltpu.async_copy(
                yb[st][b], yh[st].at[pl.ds(t0 + _CH * q, _CH)], semw[st][b])
    for st in range(2):
        for b in range(2):
            if wd[st][b] is not None:
                wd[st][b].wait()


def _sc_gather(y_sorted, pos0r, pos1r, interpret=False):
    mesh = plsc.VectorSubcoreMesh(
        core_axis_name="c", subcore_axis_name="s", num_cores=_NC, num_subcores=_NS
    )
    fn = pl.kernel(
        _sc_gather_body,
        out_type=[
            jax.ShapeDtypeStruct((T, D), jnp.float32),
            jax.ShapeDtypeStruct((T, D), jnp.float32),
        ],
        mesh=mesh,
        scratch_types=[
            pltpu.VMEM((_NQ, _CH), jnp.int32),
            pltpu.VMEM((_NQ, _CH), jnp.int32),
            pltpu.VMEM((_CH, D), jnp.float32),
            pltpu.VMEM((_CH, D), jnp.float32),
            pltpu.VMEM((_CH, D), jnp.float32),
            pltpu.VMEM((_CH, D), jnp.float32),
        ] + [pltpu.SemaphoreType.DMA] * 8,
        interpret=interpret,
    )
    return fn(y_sorted, pos0r, pos1r)


def _finish_body(x_ref, wg_ref, wu_ref, wd_ref, y0_ref, y1_ref, w0_ref, w1_ref,
                 out_ref):
    x = x_ref[...]
    g = jnp.dot(x, wg_ref[...], preferred_element_type=jnp.float32)
    u = jnp.dot(x, wu_ref[...], preferred_element_type=jnp.float32)
    h = g * jax.nn.sigmoid(g) * u
    y = jnp.dot(h, wd_ref[...], preferred_element_type=jnp.float32)
    out_ref[...] = y + w0_ref[...] * y0_ref[...] + w1_ref[...] * y1_ref[...]


def _finish(xb, wg, wu, wd, y0t, y1t, w0c, w1c, interpret=False):
    nt = T // _BT_FIN
    return pl.pallas_call(
        _finish_body,
        grid=(nt,),
        in_specs=[
            pl.BlockSpec((_BT_FIN, D), lambda i: (i, 0)),
            pl.BlockSpec((D, I), lambda i: (0, 0)),
            pl.BlockSpec((D, I), lambda i: (0, 0)),
            pl.BlockSpec((I, D), lambda i: (0, 0)),
            pl.BlockSpec((_BT_FIN, D), lambda i: (i, 0)),
            pl.BlockSpec((_BT_FIN, D), lambda i: (i, 0)),
            pl.BlockSpec((_BT_FIN, 1), lambda i: (i, 0)),
            pl.BlockSpec((_BT_FIN, 1), lambda i: (i, 0)),
        ],
        out_specs=pl.BlockSpec((_BT_FIN, D), lambda i: (i, 0)),
        out_shape=jax.ShapeDtypeStruct((T, D), jnp.float32),
        interpret=interpret,
    )(xb, wg, wu, wd, y0t, y1t, w0c, w1c)


def _kernel_impl(hidden_states, Wg_sh, Wu_sh, Wd_sh, W_router, e_bias,
                 We_gate, We_up, We_down, interpret=False):
    x = hidden_states
    pos0, pos1, w0, w1, meta = _route2(x, W_router, e_bias, interpret=interpret)
    pos0r = pos0.reshape(T // _CH, _CH)
    pos1r = pos1.reshape(T // _CH, _CH)
    x_sorted = _sc_scatter(pos0r, pos1r, x, interpret=interpret)

    y_sorted = _gmm(meta, x_sorted, We_gate, We_up, We_down, interpret=interpret)

    y0t, y1t = _sc_gather(y_sorted, pos0r, pos1r, interpret=interpret)

    out = _finish(x, Wg_sh, Wu_sh, Wd_sh, y0t, y1t,
                  w0.reshape(T, 1), w1.reshape(T, 1), interpret=interpret)
    return out


def kernel(hidden_states, Wg_sh, Wu_sh, Wd_sh, W_router, e_bias, We_gate, We_up, We_down):
    return _kernel_impl(hidden_states, Wg_sh, Wu_sh, Wd_sh, W_router, e_bias,
                        We_gate, We_up, We_down)
